# baseline (speedup 1.0000x reference)
import jax
import jax.numpy as jnp
from jax import lax
from jax.experimental import pallas as pl
from jax.experimental.pallas import tpu as pltpu
from jax.experimental.pallas import tpu_sc as plsc

N = 10000
NP = 10240
E = 160000
EP = 163840
L = 16
NC = 2
NS = 16
CH = 128
PCH = 64
SEG_PER_TILE = EP // NS
DEG_PER_TILE = EP // (NC * NS)
RPT32 = NP // (NC * NS)
RPT16 = NP // NS

_MESH = plsc.VectorSubcoreMesh(core_axis_name="c", subcore_axis_name="s")


def _sc_deg_body(dst_hbm, ones_hbm, deg_out, acc_sh, ones_v, didx, zbuf):
    c = lax.axis_index("c")
    s = lax.axis_index("s")
    wid = s * NC + c

    pltpu.sync_copy(ones_hbm, ones_v)

    def zbody(i, _):
        for k in range(128 // L):
            zbuf[i, pl.ds(k * L, L)] = jnp.zeros((L,), jnp.float32)
        return 0
    lax.fori_loop(0, CH, zbody, 0)
    for t in range(RPT16 // CH):
        pltpu.sync_copy(zbuf, acc_sh.at[pl.ds(s * RPT16 + t * CH, CH)])
    plsc.subcore_barrier()

    def deg_body(j, _):
        off = wid * DEG_PER_TILE + j * CH
        pltpu.sync_copy(dst_hbm.at[pl.ds(off, CH)], didx)
        pltpu.sync_copy(ones_v, acc_sh.at[didx], add=True)
        return 0
    lax.fori_loop(0, DEG_PER_TILE // CH, deg_body, 0)
    plsc.subcore_barrier()
    pltpu.sync_copy(acc_sh.at[pl.ds(s * RPT16, RPT16)],
                    deg_out.at[c, pl.ds(s * RPT16, RPT16)])


def _sc_deg(dstp, ones_c):
    f = pl.kernel(
        _sc_deg_body,
        out_type=jax.ShapeDtypeStruct((NC, NP, 128), jnp.float32),
        mesh=_MESH,
        scratch_types=(
            pltpu.VMEM_SHARED((NP, 128), jnp.float32),
            pltpu.VMEM((CH, 128), jnp.float32),
            pltpu.VMEM((CH,), jnp.int32),
            pltpu.VMEM((CH, 128), jnp.float32),
        ),
    )
    return f(dstp, ones_c)


def _sc_sperm_body(ys_hbm, pos_hbm, xs_out, idx2d, rows_v, sem):
    c = lax.axis_index("c")
    s = lax.axis_index("s")
    base = s * RPT16
    nch = RPT16 // PCH
    col_off = c * NP

    for j in range(nch):
        pltpu.sync_copy(pos_hbm.at[pl.ds(base + j * PCH, PCH)], idx2d.at[j])
        for k in range(PCH // L):
            sl = pl.ds(k * L, L)
            idx2d[j, sl] = idx2d[j, sl] + col_off

    for j in range(nch):
        pltpu.sync_copy(ys_hbm.at[c, pl.ds(base + j * PCH, PCH)], rows_v)
        pltpu.async_copy(rows_v, xs_out.at[idx2d.at[j]], sem).wait()


def _sc_sperm(ys, pos_flat):
    f = pl.kernel(
        _sc_sperm_body,
        out_type=jax.ShapeDtypeStruct((2 * NP, 128), jnp.float32),
        mesh=_MESH,
        scratch_types=(
            pltpu.VMEM((RPT16 // PCH, PCH), jnp.int32),
            pltpu.VMEM((PCH, 128), jnp.float32),
            pltpu.SemaphoreType.DMA,
        ),
    )
    return f(ys, pos_flat)


_NBUF = 2
_NCHT = SEG_PER_TILE // CH


def _sc_segsum_body(xs_hbm, src2_hbm, dst2_hbm, acc_out, acc_sh,
                    sidx2, didx2, *rest):
    rows = rest[0:_NBUF]
    gsem = rest[_NBUF:2 * _NBUF]
    ssem = rest[2 * _NBUF:3 * _NBUF]
    c = lax.axis_index("c")
    s = lax.axis_index("s")
    col_off = c * NP
    nh = _NCHT // 2

    def zbody(i, _):
        for b in range(_NBUF):
            for k in range(128 // L):
                rows[b][i, pl.ds(k * L, L)] = jnp.zeros((L,), jnp.float32)
        return 0
    lax.fori_loop(0, CH, zbody, 0)
    for t in range(RPT16 // CH):
        pltpu.sync_copy(rows[0], acc_sh.at[pl.ds(s * RPT16 + t * CH, CH)])
    plsc.subcore_barrier()

    for h in range(2):
        pltpu.sync_copy(src2_hbm.at[pl.ds(s * _NCHT + h * nh, nh)], sidx2)
        pltpu.sync_copy(dst2_hbm.at[pl.ds(s * _NCHT + h * nh, nh)], didx2)

        def adj(i, _):
            for k in range(CH // L):
                sl = pl.ds(k * L, L)
                sidx2[i, sl] = sidx2[i, sl] + col_off
            return 0
        lax.fori_loop(0, nh, adj, 0)

        for b in range(_NBUF):
            pltpu.async_copy(rows[b], acc_sh.at[didx2.at[b]], ssem[b],
                             add=True)

        def body(r, _):
            for b in range(_NBUF):
                j = r * _NBUF + b
                pltpu.make_async_copy(rows[b], acc_sh.at[didx2.at[0]],
                                      ssem[b]).wait()
                pltpu.async_copy(xs_hbm.at[sidx2.at[j]], rows[b], gsem[b])
            for b in range(_NBUF):
                j = r * _NBUF + b
                pltpu.make_async_copy(xs_hbm.at[sidx2.at[0]], rows[b],
                                      gsem[b]).wait()
                pltpu.async_copy(rows[b], acc_sh.at[didx2.at[j]], ssem[b],
                                 add=True)
            return 0
        lax.fori_loop(0, nh // _NBUF, body, 0)
        for b in range(_NBUF):
            pltpu.make_async_copy(rows[b], acc_sh.at[didx2.at[0]],
                                  ssem[b]).wait()
            def rz(i, _, _b=b):
                for k in range(128 // L):
                    rows[_b][i, pl.ds(k * L, L)] = jnp.zeros((L,),
                                                             jnp.float32)
                return 0
            lax.fori_loop(0, CH, rz, 0)
    plsc.subcore_barrier()

    pltpu.sync_copy(acc_sh.at[pl.ds(s * RPT16, RPT16)],
                    acc_out.at[c, pl.ds(s * RPT16, RPT16)])


def _sc_segsum(xs_flat, src2, dst2):
    f = pl.kernel(
        _sc_segsum_body,
        out_type=jax.ShapeDtypeStruct((2, NP, 128), jnp.float32),
        mesh=_MESH,
        scratch_types=(
            (pltpu.VMEM_SHARED((NP, 128), jnp.float32),
             pltpu.VMEM((_NCHT // 2, CH), jnp.int32),
             pltpu.VMEM((_NCHT // 2, CH), jnp.int32))
            + tuple(pltpu.VMEM((CH, 128), jnp.float32) for _ in range(_NBUF))
            + tuple(pltpu.SemaphoreType.DMA for _ in range(_NBUF))
            + tuple(pltpu.SemaphoreType.DMA for _ in range(_NBUF))
        ),
    )
    return f(xs_flat, src2, dst2)


_PC = 1024


def _tc_pos_body(mask_ref, pos_ref, carry):
    p = pl.program_id(0)
    j = pl.program_id(1)

    @pl.when(jnp.logical_and(p == 0, j == 0))
    def _():
        carry[0] = 0.0

    m = mask_ref[...]
    msum = jnp.sum(m)

    @pl.when(p == 0)
    def _():
        carry[0] = carry[0] + msum
        pos_ref[...] = jnp.zeros((1, _PC), jnp.int32)

    @pl.when(p == 1)
    def _():
        @pl.when(j == 0)
        def _():
            carry[1] = 0.0
        r = lax.broadcasted_iota(jnp.int32, (_PC, _PC), 0)
        cc = lax.broadcasted_iota(jnp.int32, (_PC, _PC), 1)
        ut = (r <= cc).astype(jnp.float32)
        incl = jnp.dot(m, ut, preferred_element_type=jnp.float32)
        excl = incl - m
        cum_t = carry[1] + excl
        ii = (lax.broadcasted_iota(jnp.int32, (1, _PC), 1).astype(jnp.float32)
              + jnp.float32(_PC) * j.astype(jnp.float32))
        k_tot = carry[0]
        posf = jnp.where(m > 0.5, cum_t, k_tot + ii - cum_t)
        pos_ref[...] = posf.astype(jnp.int32)
        carry[1] = carry[1] + msum


def _tc_pos(maskf_row):
    return pl.pallas_call(
        _tc_pos_body,
        grid=(2, NP // _PC),
        in_specs=[pl.BlockSpec((1, _PC), lambda p, j: (0, j))],
        out_specs=pl.BlockSpec((1, _PC), lambda p, j: (0, j)),
        out_shape=jax.ShapeDtypeStruct((1, NP), jnp.int32),
        scratch_shapes=[pltpu.SMEM((2,), jnp.float32)],
        compiler_params=pltpu.CompilerParams(
            dimension_semantics=("arbitrary", "arbitrary")),
    )(maskf_row)


_DR = 1280


def _tc_dis_body(deg_ref, dis_ref, dinv_ref):
    d = deg_ref[0, :, 0:1] + deg_ref[1, :, 0:1] + 1.0
    dis_ref[...] = lax.rsqrt(d)
    dinv_ref[...] = jnp.sqrt(d)


def _tc_dis(deg2):
    return pl.pallas_call(
        _tc_dis_body,
        grid=(NP // _DR,),
        in_specs=[pl.BlockSpec((2, _DR, 128), lambda i: (0, i, 0))],
        out_specs=(pl.BlockSpec((_DR, 1), lambda i: (i, 0)),
                   pl.BlockSpec((_DR, 1), lambda i: (i, 0))),
        out_shape=(jax.ShapeDtypeStruct((NP, 1), jnp.float32),
                   jax.ShapeDtypeStruct((NP, 1), jnp.float32)),
        compiler_params=pltpu.CompilerParams(
            dimension_semantics=("parallel",)),
    )(deg2)


def _tc_scale_body(xsu_ref, dis_ref, xs_ref):
    d = dis_ref[...]
    xs_ref[0] = xsu_ref[0] * d
    xs_ref[1] = xsu_ref[1] * d


def _tc_scale(xsu3, dis_col):
    return pl.pallas_call(
        _tc_scale_body,
        grid=(N // _LR,),
        in_specs=[pl.BlockSpec((2, _LR, 128), lambda i: (0, i, 0)),
                  pl.BlockSpec((_LR, 1), lambda i: (i, 0))],
        out_specs=pl.BlockSpec((2, _LR, 128), lambda i: (0, i, 0)),
        out_shape=jax.ShapeDtypeStruct((2, NP, 128), jnp.float32),
        compiler_params=pltpu.CompilerParams(
            dimension_semantics=("parallel",)),
    )(xsu3, dis_col)


_MR = 1000
_MK = 512
_NKT = 4096 // _MK


def _tc_mlp_body(maskf_ref, x_ref, w1_ref, b1_ref, w2_ref, b2_ref,
                 ys_ref, acc_s, gene_s):
    k = pl.program_id(1)

    @pl.when(k == 0)
    def _():
        gene_s[...] = x_ref[:, :256]

    prod = jnp.dot(x_ref[...], w1_ref[...],
                   preferred_element_type=jnp.float32)

    @pl.when(k == 0)
    def _():
        acc_s[...] = prod

    @pl.when(k > 0)
    def _():
        acc_s[...] = acc_s[...] + prod

    @pl.when(k == _NKT - 1)
    def _():
        g = jnp.maximum(acc_s[...] + b1_ref[...], 0.0)
        g = jnp.maximum(jnp.dot(g, w2_ref[...],
                                preferred_element_type=jnp.float32)
                        + b2_ref[...], 0.0)
        ys = jnp.where(maskf_ref[...] > 0.5, gene_s[...], g)
        ys_ref[0] = ys[:, :128]
        ys_ref[1] = ys[:, 128:]


def _tc_mlp(maskf, x, w1, b1r, w2, b2r):
    return pl.pallas_call(
        _tc_mlp_body,
        grid=(N // _MR, _NKT),
        in_specs=[
            pl.BlockSpec((_MR, 1), lambda i, k: (i, 0)),
            pl.BlockSpec((_MR, _MK), lambda i, k: (i, k)),
            pl.BlockSpec((_MK, 1024), lambda i, k: (k, 0)),
            pl.BlockSpec((1, 1024), lambda i, k: (0, 0)),
            pl.BlockSpec((1024, 256), lambda i, k: (0, 0)),
            pl.BlockSpec((1, 256), lambda i, k: (0, 0)),
        ],
        out_specs=pl.BlockSpec((2, _MR, 128), lambda i, k: (0, i, 0)),
        out_shape=jax.ShapeDtypeStruct((2, NP, 128), jnp.float32),
        scratch_shapes=[pltpu.VMEM((_MR, 1024), jnp.float32),
                        pltpu.VMEM((_MR, 256), jnp.float32)],
        compiler_params=pltpu.CompilerParams(
            dimension_semantics=("parallel", "arbitrary")),
    )(maskf, x, w1, b1r, w2, b2r)


_LR = 1000


def _tc_l1_body(acc_ref, xs_ref, dis_ref, dinv_ref,
                wc_ref, bc_ref, wr_ref, br_ref, out_ref):
    xlo = xs_ref[0]
    xhi = xs_ref[1]
    dis = dis_ref[...]
    agg = jnp.concatenate([acc_ref[0] + xlo, acc_ref[1] + xhi], axis=1) * dis
    xt = jnp.concatenate([xlo, xhi], axis=1) * dinv_ref[...]
    h = jnp.maximum(jnp.dot(agg, wc_ref[...],
                            preferred_element_type=jnp.float32)
                    + bc_ref[...], 0.0)
    h = h + jnp.dot(xt, wr_ref[...], preferred_element_type=jnp.float32) \
        + br_ref[...]
    xs2 = h * dis
    out_ref[0] = xs2[:, :128]
    out_ref[1] = xs2[:, 128:]


def _tc_layer1(acc3, xs3, dis_col, dinv_col, wc, bcr, wr, brr):
    return pl.pallas_call(
        _tc_l1_body,
        grid=(N // _LR,),
        in_specs=[
            pl.BlockSpec((2, _LR, 128), lambda i: (0, i, 0)),
            pl.BlockSpec((2, _LR, 128), lambda i: (0, i, 0)),
            pl.BlockSpec((_LR, 1), lambda i: (i, 0)),
            pl.BlockSpec((_LR, 1), lambda i: (i, 0)),
            pl.BlockSpec((256, 256), lambda i: (0, 0)),
            pl.BlockSpec((1, 256), lambda i: (0, 0)),
            pl.BlockSpec((256, 256), lambda i: (0, 0)),
            pl.BlockSpec((1, 256), lambda i: (0, 0)),
        ],
        out_specs=pl.BlockSpec((2, _LR, 128), lambda i: (0, i, 0)),
        out_shape=jax.ShapeDtypeStruct((2, NP, 128), jnp.float32),
        compiler_params=pltpu.CompilerParams(
            dimension_semantics=("parallel",)),
    )(acc3, xs3, dis_col, dinv_col, wc, bcr, wr, brr)


def _tc_l2_body(acc_ref, xs_ref, dis_ref, dinv_ref, istj_ref,
                wc_ref, bc_ref, wr_ref, br_ref, wf_ref, wfl_ref, bf_ref,
                out_ref):
    xlo = xs_ref[0]
    xhi = xs_ref[1]
    agg = jnp.concatenate([acc_ref[0] + xlo, acc_ref[1] + xhi],
                          axis=1) * dis_ref[...]
    h1 = jnp.concatenate([xlo, xhi], axis=1) * dinv_ref[...]
    h2 = jnp.maximum(jnp.dot(agg, wc_ref[...],
                             preferred_element_type=jnp.float32)
                     + bc_ref[...], 0.0)
    h2 = h2 + jnp.dot(h1, wr_ref[...], preferred_element_type=jnp.float32) \
        + br_ref[...]
    out_ref[...] = (jnp.dot(h2, wf_ref[...],
                            preferred_element_type=jnp.float32)
                    + istj_ref[...] * wfl_ref[...] + bf_ref[...])


def _tc_layer2(acc3, xs3, dis_col, dinv_col, istj,
               wc, bcr, wr, brr, wf256, wflast, bfr):
    return pl.pallas_call(
        _tc_l2_body,
        grid=(N // _LR,),
        in_specs=[
            pl.BlockSpec((2, _LR, 128), lambda i: (0, i, 0)),
            pl.BlockSpec((2, _LR, 128), lambda i: (0, i, 0)),
            pl.BlockSpec((_LR, 1), lambda i: (i, 0)),
            pl.BlockSpec((_LR, 1), lambda i: (i, 0)),
            pl.BlockSpec((_LR, 1), lambda i: (i, 0)),
            pl.BlockSpec((256, 256), lambda i: (0, 0)),
            pl.BlockSpec((1, 256), lambda i: (0, 0)),
            pl.BlockSpec((256, 256), lambda i: (0, 0)),
            pl.BlockSpec((1, 256), lambda i: (0, 0)),
            pl.BlockSpec((256, 128), lambda i: (0, 0)),
            pl.BlockSpec((1, 128), lambda i: (0, 0)),
            pl.BlockSpec((1, 128), lambda i: (0, 0)),
        ],
        out_specs=pl.BlockSpec((_LR, 128), lambda i: (i, 0)),
        out_shape=jax.ShapeDtypeStruct((N, 128), jnp.float32),
        compiler_params=pltpu.CompilerParams(
            dimension_semantics=("parallel",)),
    )(acc3, xs3, dis_col, dinv_col, istj, wc, bcr, wr, brr,
      wf256, wflast, bfr)


def kernel(x, edge_index, mask, istj_predict, W1, b1, W2, b2,
           Wc1, bc1, Wc2, bc2, Wr1, br1, Wr2, br2, Wf, bf):
    maskf = mask.astype(jnp.float32).reshape(N, 1)
    maskf_row = jnp.pad(mask.astype(jnp.float32), (0, NP - N)).reshape(1, NP)
    src = edge_index[0]
    dst = edge_index[1]
    srcp = jnp.concatenate([src, jnp.full((EP - E,), N, jnp.int32)])
    dstp = jnp.concatenate([dst, jnp.full((EP - E,), NP - 1, jnp.int32)])
    ones_c = jnp.ones((CH, 128), jnp.float32)

    pos_flat = _tc_pos(maskf_row).reshape(NP)
    deg2 = _sc_deg(dstp, ones_c)
    dis_col, dinv_col = _tc_dis(deg2)

    ysu = _tc_mlp(maskf, x, W1, b1.reshape(1, -1), W2, b2.reshape(1, -1))
    xsu_flat = _sc_sperm(ysu, pos_flat)
    xs3 = _tc_scale(xsu_flat.reshape(2, NP, 128), dis_col)
    src2 = srcp.reshape(EP // CH, CH)
    dst2 = dstp.reshape(EP // CH, CH)
    acc1 = _sc_segsum(xs3.reshape(2 * NP, 128), src2, dst2)
    xs2 = _tc_layer1(acc1, xs3, dis_col, dinv_col,
                     Wc1, bc1.reshape(1, -1), Wr1, br1.reshape(1, -1))
    acc2 = _sc_segsum(xs2.reshape(2 * NP, 128), src2, dst2)
    out = _tc_layer2(acc2, xs2, dis_col, dinv_col, istj_predict,
                     Wc2, bc2.reshape(1, -1), Wr2, br2.reshape(1, -1),
                     Wf[:256], Wf[256:257], bf.reshape(1, -1))
    return out


# speedup vs baseline: 1.0720x; 1.0720x over previous
import jax
import jax.numpy as jnp
from jax import lax
from jax.experimental import pallas as pl
from jax.experimental.pallas import tpu as pltpu
from jax.experimental.pallas import tpu_sc as plsc

N = 10000
NP = 10240
E = 160000
EP = 163840
L = 16
NC = 2
NS = 16
CH = 128
PCH = 64
SEG_PER_TILE = EP // NS
DEG_PER_TILE = EP // (NC * NS)
RPT32 = NP // (NC * NS)
RPT16 = NP // NS

_MESH = plsc.VectorSubcoreMesh(core_axis_name="c", subcore_axis_name="s")


def _sc_deg_body(dst_hbm, ones_hbm, deg_out, acc_sh, ones_v, didx, zbuf):
    c = lax.axis_index("c")
    s = lax.axis_index("s")
    wid = s * NC + c

    pltpu.sync_copy(ones_hbm, ones_v)

    def zbody(i, _):
        for k in range(128 // L):
            zbuf[i, pl.ds(k * L, L)] = jnp.zeros((L,), jnp.float32)
        return 0
    lax.fori_loop(0, CH, zbody, 0)
    for t in range(RPT16 // CH):
        pltpu.sync_copy(zbuf, acc_sh.at[pl.ds(s * RPT16 + t * CH, CH)])
    plsc.subcore_barrier()

    def deg_body(j, _):
        off = wid * DEG_PER_TILE + j * CH
        pltpu.sync_copy(dst_hbm.at[pl.ds(off, CH)], didx)
        pltpu.sync_copy(ones_v, acc_sh.at[didx], add=True)
        return 0
    lax.fori_loop(0, DEG_PER_TILE // CH, deg_body, 0)
    plsc.subcore_barrier()
    pltpu.sync_copy(acc_sh.at[pl.ds(s * RPT16, RPT16)],
                    deg_out.at[c, pl.ds(s * RPT16, RPT16)])


def _sc_deg(dstp, ones_c):
    f = pl.kernel(
        _sc_deg_body,
        out_type=jax.ShapeDtypeStruct((NC, NP, 128), jnp.float32),
        mesh=_MESH,
        scratch_types=(
            pltpu.VMEM_SHARED((NP, 128), jnp.float32),
            pltpu.VMEM((CH, 128), jnp.float32),
            pltpu.VMEM((CH,), jnp.int32),
            pltpu.VMEM((CH, 128), jnp.float32),
        ),
    )
    return f(dstp, ones_c)


def _sc_sperm_body(ys_hbm, pos_hbm, xs_out, idx2d, rows_v, sem):
    c = lax.axis_index("c")
    s = lax.axis_index("s")
    base = s * RPT16
    nch = RPT16 // PCH
    col_off = c * NP

    for j in range(nch):
        pltpu.sync_copy(pos_hbm.at[pl.ds(base + j * PCH, PCH)], idx2d.at[j])
        for k in range(PCH // L):
            sl = pl.ds(k * L, L)
            idx2d[j, sl] = idx2d[j, sl] + col_off

    for j in range(nch):
        pltpu.sync_copy(ys_hbm.at[c, pl.ds(base + j * PCH, PCH)], rows_v)
        pltpu.async_copy(rows_v, xs_out.at[idx2d.at[j]], sem).wait()


def _sc_sperm(ys, pos_flat):
    f = pl.kernel(
        _sc_sperm_body,
        out_type=jax.ShapeDtypeStruct((2 * NP, 128), jnp.float32),
        mesh=_MESH,
        scratch_types=(
            pltpu.VMEM((RPT16 // PCH, PCH), jnp.int32),
            pltpu.VMEM((PCH, 128), jnp.float32),
            pltpu.SemaphoreType.DMA,
        ),
    )
    return f(ys, pos_flat)


_NBUF = 3
_ACCR = 10112
_NCHT = SEG_PER_TILE // CH


def _sc_segsum_body(xs_hbm, src_hbm, dst_hbm, acc_out, acc_sh, *rest):
    sidx = rest[0:_NBUF]
    didx = rest[_NBUF:2 * _NBUF]
    rows = rest[2 * _NBUF:3 * _NBUF]
    gsem = rest[3 * _NBUF:4 * _NBUF]
    ssem = rest[4 * _NBUF:5 * _NBUF]
    c = lax.axis_index("c")
    s = lax.axis_index("s")
    col_off = c * NP

    def zbody(i, _):
        for b in range(_NBUF):
            for k in range(128 // L):
                rows[b][i, pl.ds(k * L, L)] = jnp.zeros((L,), jnp.float32)
        return 0
    lax.fori_loop(0, CH, zbody, 0)
    for b in range(_NBUF):
        for k in range(CH // L):
            didx[b][pl.ds(k * L, L)] = jnp.zeros((L,), jnp.int32)
    abase = s * (_ACCR // NS)
    for t in range(4):
        pltpu.sync_copy(rows[0], acc_sh.at[pl.ds(abase + t * CH, CH)])
    pltpu.sync_copy(rows[0].at[pl.ds(0, 120)],
                    acc_sh.at[pl.ds(abase + 4 * CH, 120)])
    plsc.subcore_barrier()

    for b in range(_NBUF):
        pltpu.async_copy(rows[b], acc_sh.at[didx[b]], ssem[b], add=True)

    def step(j, b):
        off = s * SEG_PER_TILE + j * CH
        pltpu.make_async_copy(rows[b], acc_sh.at[didx[b]], ssem[b]).wait()
        pltpu.sync_copy(src_hbm.at[pl.ds(off, CH)], sidx[b])
        pltpu.sync_copy(dst_hbm.at[pl.ds(off, CH)], didx[b])
        for k in range(CH // L):
            sl = pl.ds(k * L, L)
            sidx[b][sl] = sidx[b][sl] + col_off
        pltpu.async_copy(xs_hbm.at[sidx[b]], rows[b], gsem[b])

    def drain(b):
        pltpu.make_async_copy(xs_hbm.at[sidx[b]], rows[b], gsem[b]).wait()
        pltpu.async_copy(rows[b], acc_sh.at[didx[b]], ssem[b], add=True)

    nfull = _NCHT // _NBUF
    ntail = _NCHT - nfull * _NBUF

    def body(r, _):
        for b in range(_NBUF):
            step(r * _NBUF + b, b)
        for b in range(_NBUF):
            drain(b)
        return 0
    lax.fori_loop(0, nfull, body, 0)
    for b in range(ntail):
        step(nfull * _NBUF + b, b)
    for b in range(ntail):
        drain(b)
    for b in range(_NBUF):
        pltpu.make_async_copy(rows[b], acc_sh.at[didx[b]], ssem[b]).wait()
    plsc.subcore_barrier()

    for t in range(4):
        pltpu.sync_copy(acc_sh.at[pl.ds(abase + t * CH, CH)],
                        acc_out.at[c, pl.ds(abase + t * CH, CH)])
    pltpu.sync_copy(acc_sh.at[pl.ds(abase + 4 * CH, 120)],
                    acc_out.at[c, pl.ds(abase + 4 * CH, 120)])


def _sc_segsum(xs_flat, srcp, dstp):
    f = pl.kernel(
        _sc_segsum_body,
        out_type=jax.ShapeDtypeStruct((2, NP, 128), jnp.float32),
        mesh=_MESH,
        scratch_types=(
            (pltpu.VMEM_SHARED((_ACCR, 128), jnp.float32),)
            + tuple(pltpu.VMEM((CH,), jnp.int32) for _ in range(_NBUF))
            + tuple(pltpu.VMEM((CH,), jnp.int32) for _ in range(_NBUF))
            + tuple(pltpu.VMEM((CH, 128), jnp.float32) for _ in range(_NBUF))
            + tuple(pltpu.SemaphoreType.DMA for _ in range(_NBUF))
            + tuple(pltpu.SemaphoreType.DMA for _ in range(_NBUF))
        ),
    )
    return f(xs_flat, srcp, dstp)


_PC = 1024


def _tc_pos_body(mask_ref, pos_ref, carry):
    p = pl.program_id(0)
    j = pl.program_id(1)

    @pl.when(jnp.logical_and(p == 0, j == 0))
    def _():
        carry[0] = 0.0

    m = mask_ref[...]
    msum = jnp.sum(m)

    @pl.when(p == 0)
    def _():
        carry[0] = carry[0] + msum
        pos_ref[...] = jnp.zeros((1, _PC), jnp.int32)

    @pl.when(p == 1)
    def _():
        @pl.when(j == 0)
        def _():
            carry[1] = 0.0
        r = lax.broadcasted_iota(jnp.int32, (_PC, _PC), 0)
        cc = lax.broadcasted_iota(jnp.int32, (_PC, _PC), 1)
        ut = (r <= cc).astype(jnp.float32)
        incl = jnp.dot(m, ut, preferred_element_type=jnp.float32)
        excl = incl - m
        cum_t = carry[1] + excl
        ii = (lax.broadcasted_iota(jnp.int32, (1, _PC), 1).astype(jnp.float32)
              + jnp.float32(_PC) * j.astype(jnp.float32))
        k_tot = carry[0]
        posf = jnp.where(m > 0.5, cum_t, k_tot + ii - cum_t)
        pos_ref[...] = posf.astype(jnp.int32)
        carry[1] = carry[1] + msum


def _tc_pos(maskf_row):
    return pl.pallas_call(
        _tc_pos_body,
        grid=(2, NP // _PC),
        in_specs=[pl.BlockSpec((1, _PC), lambda p, j: (0, j))],
        out_specs=pl.BlockSpec((1, _PC), lambda p, j: (0, j)),
        out_shape=jax.ShapeDtypeStruct((1, NP), jnp.int32),
        scratch_shapes=[pltpu.SMEM((2,), jnp.float32)],
        compiler_params=pltpu.CompilerParams(
            dimension_semantics=("arbitrary", "arbitrary")),
    )(maskf_row)


_DR = 1280


def _tc_dis_body(deg_ref, dis_ref, dinv_ref):
    d = deg_ref[0, :, 0:1] + deg_ref[1, :, 0:1] + 1.0
    dis_ref[...] = lax.rsqrt(d)
    dinv_ref[...] = jnp.sqrt(d)


def _tc_dis(deg2):
    return pl.pallas_call(
        _tc_dis_body,
        grid=(NP // _DR,),
        in_specs=[pl.BlockSpec((2, _DR, 128), lambda i: (0, i, 0))],
        out_specs=(pl.BlockSpec((_DR, 1), lambda i: (i, 0)),
                   pl.BlockSpec((_DR, 1), lambda i: (i, 0))),
        out_shape=(jax.ShapeDtypeStruct((NP, 1), jnp.float32),
                   jax.ShapeDtypeStruct((NP, 1), jnp.float32)),
        compiler_params=pltpu.CompilerParams(
            dimension_semantics=("parallel",)),
    )(deg2)


def _tc_scale_body(xsu_ref, dis_ref, xs_ref):
    d = dis_ref[...]
    xs_ref[0] = xsu_ref[0] * d
    xs_ref[1] = xsu_ref[1] * d


def _tc_scale(xsu3, dis_col):
    return pl.pallas_call(
        _tc_scale_body,
        grid=(N // _LR,),
        in_specs=[pl.BlockSpec((2, _LR, 128), lambda i: (0, i, 0)),
                  pl.BlockSpec((_LR, 1), lambda i: (i, 0))],
        out_specs=pl.BlockSpec((2, _LR, 128), lambda i: (0, i, 0)),
        out_shape=jax.ShapeDtypeStruct((2, NP, 128), jnp.float32),
        compiler_params=pltpu.CompilerParams(
            dimension_semantics=("parallel",)),
    )(xsu3, dis_col)


_MR = 1000
_MK = 512
_NKT = 4096 // _MK


def _tc_mlp_body(maskf_ref, x_ref, w1_ref, b1_ref, w2_ref, b2_ref,
                 ys_ref, acc_s, gene_s):
    k = pl.program_id(1)

    @pl.when(k == 0)
    def _():
        gene_s[...] = x_ref[:, :256]

    prod = jnp.dot(x_ref[...], w1_ref[...],
                   preferred_element_type=jnp.float32)

    @pl.when(k == 0)
    def _():
        acc_s[...] = prod

    @pl.when(k > 0)
    def _():
        acc_s[...] = acc_s[...] + prod

    @pl.when(k == _NKT - 1)
    def _():
        g = jnp.maximum(acc_s[...] + b1_ref[...], 0.0)
        g = jnp.maximum(jnp.dot(g, w2_ref[...],
                                preferred_element_type=jnp.float32)
                        + b2_ref[...], 0.0)
        ys = jnp.where(maskf_ref[...] > 0.5, gene_s[...], g)
        ys_ref[0] = ys[:, :128]
        ys_ref[1] = ys[:, 128:]


def _tc_mlp(maskf, x, w1, b1r, w2, b2r):
    return pl.pallas_call(
        _tc_mlp_body,
        grid=(N // _MR, _NKT),
        in_specs=[
            pl.BlockSpec((_MR, 1), lambda i, k: (i, 0)),
            pl.BlockSpec((_MR, _MK), lambda i, k: (i, k)),
            pl.BlockSpec((_MK, 1024), lambda i, k: (k, 0)),
            pl.BlockSpec((1, 1024), lambda i, k: (0, 0)),
            pl.BlockSpec((1024, 256), lambda i, k: (0, 0)),
            pl.BlockSpec((1, 256), lambda i, k: (0, 0)),
        ],
        out_specs=pl.BlockSpec((2, _MR, 128), lambda i, k: (0, i, 0)),
        out_shape=jax.ShapeDtypeStruct((2, NP, 128), jnp.float32),
        scratch_shapes=[pltpu.VMEM((_MR, 1024), jnp.float32),
                        pltpu.VMEM((_MR, 256), jnp.float32)],
        compiler_params=pltpu.CompilerParams(
            dimension_semantics=("parallel", "arbitrary")),
    )(maskf, x, w1, b1r, w2, b2r)


_LR = 1000


def _tc_l1_body(acc_ref, xs_ref, dis_ref, dinv_ref,
                wc_ref, bc_ref, wr_ref, br_ref, out_ref):
    xlo = xs_ref[0]
    xhi = xs_ref[1]
    dis = dis_ref[...]
    agg = jnp.concatenate([acc_ref[0] + xlo, acc_ref[1] + xhi], axis=1) * dis
    xt = jnp.concatenate([xlo, xhi], axis=1) * dinv_ref[...]
    h = jnp.maximum(jnp.dot(agg, wc_ref[...],
                            preferred_element_type=jnp.float32)
                    + bc_ref[...], 0.0)
    h = h + jnp.dot(xt, wr_ref[...], preferred_element_type=jnp.float32) \
        + br_ref[...]
    xs2 = h * dis
    out_ref[0] = xs2[:, :128]
    out_ref[1] = xs2[:, 128:]


def _tc_layer1(acc3, xs3, dis_col, dinv_col, wc, bcr, wr, brr):
    return pl.pallas_call(
        _tc_l1_body,
        grid=(N // _LR,),
        in_specs=[
            pl.BlockSpec((2, _LR, 128), lambda i: (0, i, 0)),
            pl.BlockSpec((2, _LR, 128), lambda i: (0, i, 0)),
            pl.BlockSpec((_LR, 1), lambda i: (i, 0)),
            pl.BlockSpec((_LR, 1), lambda i: (i, 0)),
            pl.BlockSpec((256, 256), lambda i: (0, 0)),
            pl.BlockSpec((1, 256), lambda i: (0, 0)),
            pl.BlockSpec((256, 256), lambda i: (0, 0)),
            pl.BlockSpec((1, 256), lambda i: (0, 0)),
        ],
        out_specs=pl.BlockSpec((2, _LR, 128), lambda i: (0, i, 0)),
        out_shape=jax.ShapeDtypeStruct((2, NP, 128), jnp.float32),
        compiler_params=pltpu.CompilerParams(
            dimension_semantics=("parallel",)),
    )(acc3, xs3, dis_col, dinv_col, wc, bcr, wr, brr)


def _tc_l2_body(acc_ref, xs_ref, dis_ref, dinv_ref, istj_ref,
                wc_ref, bc_ref, wr_ref, br_ref, wf_ref, wfl_ref, bf_ref,
                out_ref):
    xlo = xs_ref[0]
    xhi = xs_ref[1]
    agg = jnp.concatenate([acc_ref[0] + xlo, acc_ref[1] + xhi],
                          axis=1) * dis_ref[...]
    h1 = jnp.concatenate([xlo, xhi], axis=1) * dinv_ref[...]
    h2 = jnp.maximum(jnp.dot(agg, wc_ref[...],
                             preferred_element_type=jnp.float32)
                     + bc_ref[...], 0.0)
    h2 = h2 + jnp.dot(h1, wr_ref[...], preferred_element_type=jnp.float32) \
        + br_ref[...]
    out_ref[...] = (jnp.dot(h2, wf_ref[...],
                            preferred_element_type=jnp.float32)
                    + istj_ref[...] * wfl_ref[...] + bf_ref[...])


def _tc_layer2(acc3, xs3, dis_col, dinv_col, istj,
               wc, bcr, wr, brr, wf256, wflast, bfr):
    return pl.pallas_call(
        _tc_l2_body,
        grid=(N // _LR,),
        in_specs=[
            pl.BlockSpec((2, _LR, 128), lambda i: (0, i, 0)),
            pl.BlockSpec((2, _LR, 128), lambda i: (0, i, 0)),
            pl.BlockSpec((_LR, 1), lambda i: (i, 0)),
            pl.BlockSpec((_LR, 1), lambda i: (i, 0)),
            pl.BlockSpec((_LR, 1), lambda i: (i, 0)),
            pl.BlockSpec((256, 256), lambda i: (0, 0)),
            pl.BlockSpec((1, 256), lambda i: (0, 0)),
            pl.BlockSpec((256, 256), lambda i: (0, 0)),
            pl.BlockSpec((1, 256), lambda i: (0, 0)),
            pl.BlockSpec((256, 128), lambda i: (0, 0)),
            pl.BlockSpec((1, 128), lambda i: (0, 0)),
            pl.BlockSpec((1, 128), lambda i: (0, 0)),
        ],
        out_specs=pl.BlockSpec((_LR, 128), lambda i: (i, 0)),
        out_shape=jax.ShapeDtypeStruct((N, 128), jnp.float32),
        compiler_params=pltpu.CompilerParams(
            dimension_semantics=("parallel",)),
    )(acc3, xs3, dis_col, dinv_col, istj, wc, bcr, wr, brr,
      wf256, wflast, bfr)


def kernel(x, edge_index, mask, istj_predict, W1, b1, W2, b2,
           Wc1, bc1, Wc2, bc2, Wr1, br1, Wr2, br2, Wf, bf):
    maskf = mask.astype(jnp.float32).reshape(N, 1)
    maskf_row = jnp.pad(mask.astype(jnp.float32), (0, NP - N)).reshape(1, NP)
    src = edge_index[0]
    dst = edge_index[1]
    srcp = jnp.concatenate([src, jnp.full((EP - E,), N, jnp.int32)])
    dstp = jnp.concatenate([dst, jnp.full((EP - E,), 10104, jnp.int32)])
    ones_c = jnp.ones((CH, 128), jnp.float32)

    pos_flat = _tc_pos(maskf_row).reshape(NP)
    deg2 = _sc_deg(dstp, ones_c)
    dis_col, dinv_col = _tc_dis(deg2)

    ysu = _tc_mlp(maskf, x, W1, b1.reshape(1, -1), W2, b2.reshape(1, -1))
    xsu_flat = _sc_sperm(ysu, pos_flat)
    xs3 = _tc_scale(xsu_flat.reshape(2, NP, 128), dis_col)
    acc1 = _sc_segsum(xs3.reshape(2 * NP, 128), srcp, dstp)
    xs2 = _tc_layer1(acc1, xs3, dis_col, dinv_col,
                     Wc1, bc1.reshape(1, -1), Wr1, br1.reshape(1, -1))
    acc2 = _sc_segsum(xs2.reshape(2 * NP, 128), srcp, dstp)
    out = _tc_layer2(acc2, xs2, dis_col, dinv_col, istj_predict,
                     Wc2, bc2.reshape(1, -1), Wr2, br2.reshape(1, -1),
                     Wf[:256], Wf[256:257], bf.reshape(1, -1))
    return out


# speedup vs baseline: 1.1368x; 1.0605x over previous
import jax
import jax.numpy as jnp
from jax import lax
from jax.experimental import pallas as pl
from jax.experimental.pallas import tpu as pltpu
from jax.experimental.pallas import tpu_sc as plsc

N = 10000
NP = 10240
E = 160000
EP = 163840
L = 16
NC = 2
NS = 16
CH = 128
PCH = 64
SEG_PER_TILE = EP // NS
DEG_PER_TILE = EP // (NC * NS)
RPT32 = NP // (NC * NS)
RPT16 = NP // NS

_MESH = plsc.VectorSubcoreMesh(core_axis_name="c", subcore_axis_name="s")


def _sc_deg_body(dst_hbm, ones_hbm, deg_out, acc_sh, ones_v, didx, zbuf):
    c = lax.axis_index("c")
    s = lax.axis_index("s")
    wid = s * NC + c

    pltpu.sync_copy(ones_hbm, ones_v)

    def zbody(i, _):
        for k in range(128 // L):
            zbuf[i, pl.ds(k * L, L)] = jnp.zeros((L,), jnp.float32)
        return 0
    lax.fori_loop(0, CH, zbody, 0)
    for t in range(RPT16 // CH):
        pltpu.sync_copy(zbuf, acc_sh.at[pl.ds(s * RPT16 + t * CH, CH)])
    plsc.subcore_barrier()

    def deg_body(j, _):
        off = wid * DEG_PER_TILE + j * CH
        pltpu.sync_copy(dst_hbm.at[pl.ds(off, CH)], didx)
        pltpu.sync_copy(ones_v, acc_sh.at[didx], add=True)
        return 0
    lax.fori_loop(0, DEG_PER_TILE // CH, deg_body, 0)
    plsc.subcore_barrier()
    pltpu.sync_copy(acc_sh.at[pl.ds(s * RPT16, RPT16)],
                    deg_out.at[c, pl.ds(s * RPT16, RPT16)])


def _sc_deg(dstp, ones_c):
    f = pl.kernel(
        _sc_deg_body,
        out_type=jax.ShapeDtypeStruct((NC, NP, 128), jnp.float32),
        mesh=_MESH,
        scratch_types=(
            pltpu.VMEM_SHARED((NP, 128), jnp.float32),
            pltpu.VMEM((CH, 128), jnp.float32),
            pltpu.VMEM((CH,), jnp.int32),
            pltpu.VMEM((CH, 128), jnp.float32),
        ),
    )
    return f(dstp, ones_c)


def _sc_sperm_body(ys_hbm, pos_hbm, xs_out, idx2d, rows_v, sem):
    c = lax.axis_index("c")
    s = lax.axis_index("s")
    base = s * RPT16
    nch = RPT16 // PCH
    col_off = c * NP

    for j in range(nch):
        pltpu.sync_copy(pos_hbm.at[pl.ds(base + j * PCH, PCH)], idx2d.at[j])
        for k in range(PCH // L):
            sl = pl.ds(k * L, L)
            idx2d[j, sl] = idx2d[j, sl] + col_off

    for j in range(nch):
        pltpu.sync_copy(ys_hbm.at[c, pl.ds(base + j * PCH, PCH)], rows_v)
        pltpu.async_copy(rows_v, xs_out.at[idx2d.at[j]], sem).wait()


def _sc_sperm(ys, pos_flat):
    f = pl.kernel(
        _sc_sperm_body,
        out_type=jax.ShapeDtypeStruct((2 * NP, 128), jnp.float32),
        mesh=_MESH,
        scratch_types=(
            pltpu.VMEM((RPT16 // PCH, PCH), jnp.int32),
            pltpu.VMEM((PCH, 128), jnp.float32),
            pltpu.SemaphoreType.DMA,
        ),
    )
    return f(ys, pos_flat)


_NBUF = 3
_ACCR = 10112
_NCHT = SEG_PER_TILE // CH


def _sc_segsum_body(xs_hbm, src_hbm, dst_hbm, acc_out, acc_sh, *rest):
    sidx = rest[0:_NBUF]
    didx = rest[_NBUF:2 * _NBUF]
    rows = rest[2 * _NBUF:3 * _NBUF]
    gsem = rest[3 * _NBUF:4 * _NBUF]
    ssem = rest[4 * _NBUF:5 * _NBUF]
    c = lax.axis_index("c")
    s = lax.axis_index("s")
    col_off = c * NP

    def zbody(i, _):
        for b in range(_NBUF):
            for k in range(128 // L):
                rows[b][i, pl.ds(k * L, L)] = jnp.zeros((L,), jnp.float32)
        return 0
    lax.fori_loop(0, CH, zbody, 0)
    for b in range(_NBUF):
        for k in range(CH // L):
            didx[b][pl.ds(k * L, L)] = jnp.zeros((L,), jnp.int32)
    abase = s * (_ACCR // NS)
    for t in range(4):
        pltpu.sync_copy(rows[0], acc_sh.at[pl.ds(abase + t * CH, CH)])
    pltpu.sync_copy(rows[0].at[pl.ds(0, 120)],
                    acc_sh.at[pl.ds(abase + 4 * CH, 120)])
    plsc.subcore_barrier()

    for b in range(_NBUF):
        pltpu.async_copy(rows[b], acc_sh.at[didx[b]], ssem[b], add=True)

    def step(j, b):
        off = s * SEG_PER_TILE + j * CH
        pltpu.make_async_copy(rows[b], acc_sh.at[didx[b]], ssem[b]).wait()
        pltpu.sync_copy(src_hbm.at[pl.ds(off, CH)], sidx[b])
        pltpu.sync_copy(dst_hbm.at[pl.ds(off, CH)], didx[b])
        for k in range(CH // L):
            sl = pl.ds(k * L, L)
            sidx[b][sl] = sidx[b][sl] + col_off
        pltpu.async_copy(xs_hbm.at[sidx[b]], rows[b], gsem[b])

    def drain(b):
        pltpu.make_async_copy(xs_hbm.at[sidx[b]], rows[b], gsem[b]).wait()
        pltpu.async_copy(rows[b], acc_sh.at[didx[b]], ssem[b], add=True)

    nfull = _NCHT // _NBUF
    ntail = _NCHT - nfull * _NBUF

    def body(r, _):
        for b in range(_NBUF):
            step(r * _NBUF + b, b)
        for b in range(_NBUF):
            drain(b)
        return 0
    lax.fori_loop(0, nfull, body, 0)
    for b in range(ntail):
        step(nfull * _NBUF + b, b)
    for b in range(ntail):
        drain(b)
    for b in range(_NBUF):
        pltpu.make_async_copy(rows[b], acc_sh.at[didx[b]], ssem[b]).wait()
    plsc.subcore_barrier()

    for t in range(4):
        pltpu.sync_copy(acc_sh.at[pl.ds(abase + t * CH, CH)],
                        acc_out.at[c, pl.ds(abase + t * CH, CH)])
    pltpu.sync_copy(acc_sh.at[pl.ds(abase + 4 * CH, 120)],
                    acc_out.at[c, pl.ds(abase + 4 * CH, 120)])


def _sc_segsum(xs_flat, srcp, dstp):
    f = pl.kernel(
        _sc_segsum_body,
        out_type=jax.ShapeDtypeStruct((2, NP, 128), jnp.float32),
        mesh=_MESH,
        scratch_types=(
            (pltpu.VMEM_SHARED((_ACCR, 128), jnp.float32),)
            + tuple(pltpu.VMEM((CH,), jnp.int32) for _ in range(_NBUF))
            + tuple(pltpu.VMEM((CH,), jnp.int32) for _ in range(_NBUF))
            + tuple(pltpu.VMEM((CH, 128), jnp.float32) for _ in range(_NBUF))
            + tuple(pltpu.SemaphoreType.DMA for _ in range(_NBUF))
            + tuple(pltpu.SemaphoreType.DMA for _ in range(_NBUF))
        ),
    )
    return f(xs_flat, srcp, dstp)


_PC = 1024


def _tc_pos_body(mask_ref, pos_ref, carry):
    p = pl.program_id(0)
    j = pl.program_id(1)

    @pl.when(jnp.logical_and(p == 0, j == 0))
    def _():
        carry[0] = 0.0

    m = mask_ref[...]
    msum = jnp.sum(m)

    @pl.when(p == 0)
    def _():
        carry[0] = carry[0] + msum
        pos_ref[...] = jnp.zeros((1, _PC), jnp.int32)

    @pl.when(p == 1)
    def _():
        @pl.when(j == 0)
        def _():
            carry[1] = 0.0
        r = lax.broadcasted_iota(jnp.int32, (_PC, _PC), 0)
        cc = lax.broadcasted_iota(jnp.int32, (_PC, _PC), 1)
        ut = (r <= cc).astype(jnp.float32)
        incl = jnp.dot(m, ut, preferred_element_type=jnp.float32)
        excl = incl - m
        cum_t = carry[1] + excl
        ii = (lax.broadcasted_iota(jnp.int32, (1, _PC), 1).astype(jnp.float32)
              + jnp.float32(_PC) * j.astype(jnp.float32))
        k_tot = carry[0]
        posf = jnp.where(m > 0.5, cum_t, k_tot + ii - cum_t)
        pos_ref[...] = posf.astype(jnp.int32)
        carry[1] = carry[1] + msum


def _tc_pos(maskf_row):
    return pl.pallas_call(
        _tc_pos_body,
        grid=(2, NP // _PC),
        in_specs=[pl.BlockSpec((1, _PC), lambda p, j: (0, j))],
        out_specs=pl.BlockSpec((1, _PC), lambda p, j: (0, j)),
        out_shape=jax.ShapeDtypeStruct((1, NP), jnp.int32),
        scratch_shapes=[pltpu.SMEM((2,), jnp.float32)],
        compiler_params=pltpu.CompilerParams(
            dimension_semantics=("arbitrary", "arbitrary")),
    )(maskf_row)


_DR = 1280


def _tc_dis_body(deg_ref, dis_ref, dinv_ref):
    d = deg_ref[0, :, 0:1] + deg_ref[1, :, 0:1] + 1.0
    dis_ref[...] = lax.rsqrt(d)
    dinv_ref[...] = jnp.sqrt(d)


def _tc_dis(deg2):
    return pl.pallas_call(
        _tc_dis_body,
        grid=(NP // _DR,),
        in_specs=[pl.BlockSpec((2, _DR, 128), lambda i: (0, i, 0))],
        out_specs=(pl.BlockSpec((_DR, 1), lambda i: (i, 0)),
                   pl.BlockSpec((_DR, 1), lambda i: (i, 0))),
        out_shape=(jax.ShapeDtypeStruct((NP, 1), jnp.float32),
                   jax.ShapeDtypeStruct((NP, 1), jnp.float32)),
        compiler_params=pltpu.CompilerParams(
            dimension_semantics=("parallel",)),
    )(deg2)


def _tc_scale_body(xsu_ref, dis_ref, xs_ref):
    d = dis_ref[...]
    xs_ref[0] = xsu_ref[0] * d
    xs_ref[1] = xsu_ref[1] * d


def _tc_scale(xsu3, dis_col):
    return pl.pallas_call(
        _tc_scale_body,
        grid=(N // _LR,),
        in_specs=[pl.BlockSpec((2, _LR, 128), lambda i: (0, i, 0)),
                  pl.BlockSpec((_LR, 1), lambda i: (i, 0))],
        out_specs=pl.BlockSpec((2, _LR, 128), lambda i: (0, i, 0)),
        out_shape=jax.ShapeDtypeStruct((2, NP, 128), jnp.float32),
        compiler_params=pltpu.CompilerParams(
            dimension_semantics=("parallel",)),
    )(xsu3, dis_col)


_MR = 1000
_MK = 512
_NKT = 4096 // _MK


def _tc_mlp_body(maskf_ref, x_ref, w1_ref, b1_ref, w2_ref, b2_ref,
                 ys_ref, acc_s, gene_s):
    k = pl.program_id(1)

    @pl.when(k == 0)
    def _():
        gene_s[...] = x_ref[:, :256]

    prod = jnp.dot(x_ref[...].astype(jnp.bfloat16), w1_ref[...],
                   preferred_element_type=jnp.float32)

    @pl.when(k == 0)
    def _():
        acc_s[...] = prod

    @pl.when(k > 0)
    def _():
        acc_s[...] = acc_s[...] + prod

    @pl.when(k == _NKT - 1)
    def _():
        g = jnp.maximum(acc_s[...] + b1_ref[...], 0.0)
        g = jnp.maximum(jnp.dot(g, w2_ref[...],
                                preferred_element_type=jnp.float32)
                        + b2_ref[...], 0.0)
        ys = jnp.where(maskf_ref[...] > 0.5, gene_s[...], g)
        ys_ref[0] = ys[:, :128]
        ys_ref[1] = ys[:, 128:]


def _tc_mlp(maskf, x, w1, b1r, w2, b2r):
    return pl.pallas_call(
        _tc_mlp_body,
        grid=(N // _MR, _NKT),
        in_specs=[
            pl.BlockSpec((_MR, 1), lambda i, k: (i, 0)),
            pl.BlockSpec((_MR, _MK), lambda i, k: (i, k)),
            pl.BlockSpec((_MK, 1024), lambda i, k: (k, 0)),
            pl.BlockSpec((1, 1024), lambda i, k: (0, 0)),
            pl.BlockSpec((1024, 256), lambda i, k: (0, 0)),
            pl.BlockSpec((1, 256), lambda i, k: (0, 0)),
        ],
        out_specs=pl.BlockSpec((2, _MR, 128), lambda i, k: (0, i, 0)),
        out_shape=jax.ShapeDtypeStruct((2, NP, 128), jnp.float32),
        scratch_shapes=[pltpu.VMEM((_MR, 1024), jnp.float32),
                        pltpu.VMEM((_MR, 256), jnp.float32)],
        compiler_params=pltpu.CompilerParams(
            dimension_semantics=("parallel", "arbitrary")),
    )(maskf, x, w1, b1r, w2, b2r)


_LR = 1000


def _tc_l1_body(acc_ref, xs_ref, dis_ref, dinv_ref,
                wc_ref, bc_ref, wr_ref, br_ref, out_ref):
    xlo = xs_ref[0]
    xhi = xs_ref[1]
    dis = dis_ref[...]
    agg = jnp.concatenate([acc_ref[0] + xlo, acc_ref[1] + xhi], axis=1) * dis
    xt = jnp.concatenate([xlo, xhi], axis=1) * dinv_ref[...]
    h = jnp.maximum(jnp.dot(agg, wc_ref[...],
                            preferred_element_type=jnp.float32)
                    + bc_ref[...], 0.0)
    h = h + jnp.dot(xt, wr_ref[...], preferred_element_type=jnp.float32) \
        + br_ref[...]
    xs2 = h * dis
    out_ref[0] = xs2[:, :128]
    out_ref[1] = xs2[:, 128:]


def _tc_layer1(acc3, xs3, dis_col, dinv_col, wc, bcr, wr, brr):
    return pl.pallas_call(
        _tc_l1_body,
        grid=(N // _LR,),
        in_specs=[
            pl.BlockSpec((2, _LR, 128), lambda i: (0, i, 0)),
            pl.BlockSpec((2, _LR, 128), lambda i: (0, i, 0)),
            pl.BlockSpec((_LR, 1), lambda i: (i, 0)),
            pl.BlockSpec((_LR, 1), lambda i: (i, 0)),
            pl.BlockSpec((256, 256), lambda i: (0, 0)),
            pl.BlockSpec((1, 256), lambda i: (0, 0)),
            pl.BlockSpec((256, 256), lambda i: (0, 0)),
            pl.BlockSpec((1, 256), lambda i: (0, 0)),
        ],
        out_specs=pl.BlockSpec((2, _LR, 128), lambda i: (0, i, 0)),
        out_shape=jax.ShapeDtypeStruct((2, NP, 128), jnp.float32),
        compiler_params=pltpu.CompilerParams(
            dimension_semantics=("parallel",)),
    )(acc3, xs3, dis_col, dinv_col, wc, bcr, wr, brr)


def _tc_l2_body(acc_ref, xs_ref, dis_ref, dinv_ref, istj_ref,
                wc_ref, bc_ref, wr_ref, br_ref, wf_ref, wfl_ref, bf_ref,
                out_ref):
    xlo = xs_ref[0]
    xhi = xs_ref[1]
    agg = jnp.concatenate([acc_ref[0] + xlo, acc_ref[1] + xhi],
                          axis=1) * dis_ref[...]
    h1 = jnp.concatenate([xlo, xhi], axis=1) * dinv_ref[...]
    h2 = jnp.maximum(jnp.dot(agg, wc_ref[...],
                             preferred_element_type=jnp.float32)
                     + bc_ref[...], 0.0)
    h2 = h2 + jnp.dot(h1, wr_ref[...], preferred_element_type=jnp.float32) \
        + br_ref[...]
    out_ref[...] = (jnp.dot(h2, wf_ref[...],
                            preferred_element_type=jnp.float32)
                    + istj_ref[...] * wfl_ref[...] + bf_ref[...])


def _tc_layer2(acc3, xs3, dis_col, dinv_col, istj,
               wc, bcr, wr, brr, wf256, wflast, bfr):
    return pl.pallas_call(
        _tc_l2_body,
        grid=(N // _LR,),
        in_specs=[
            pl.BlockSpec((2, _LR, 128), lambda i: (0, i, 0)),
            pl.BlockSpec((2, _LR, 128), lambda i: (0, i, 0)),
            pl.BlockSpec((_LR, 1), lambda i: (i, 0)),
            pl.BlockSpec((_LR, 1), lambda i: (i, 0)),
            pl.BlockSpec((_LR, 1), lambda i: (i, 0)),
            pl.BlockSpec((256, 256), lambda i: (0, 0)),
            pl.BlockSpec((1, 256), lambda i: (0, 0)),
            pl.BlockSpec((256, 256), lambda i: (0, 0)),
            pl.BlockSpec((1, 256), lambda i: (0, 0)),
            pl.BlockSpec((256, 128), lambda i: (0, 0)),
            pl.BlockSpec((1, 128), lambda i: (0, 0)),
            pl.BlockSpec((1, 128), lambda i: (0, 0)),
        ],
        out_specs=pl.BlockSpec((_LR, 128), lambda i: (i, 0)),
        out_shape=jax.ShapeDtypeStruct((N, 128), jnp.float32),
        compiler_params=pltpu.CompilerParams(
            dimension_semantics=("parallel",)),
    )(acc3, xs3, dis_col, dinv_col, istj, wc, bcr, wr, brr,
      wf256, wflast, bfr)


def kernel(x, edge_index, mask, istj_predict, W1, b1, W2, b2,
           Wc1, bc1, Wc2, bc2, Wr1, br1, Wr2, br2, Wf, bf):
    maskf = mask.astype(jnp.float32).reshape(N, 1)
    maskf_row = jnp.pad(mask.astype(jnp.float32), (0, NP - N)).reshape(1, NP)
    src = edge_index[0]
    dst = edge_index[1]
    srcp = jnp.concatenate([src, jnp.full((EP - E,), N, jnp.int32)])
    dstp = jnp.concatenate([dst, jnp.full((EP - E,), 10104, jnp.int32)])
    ones_c = jnp.ones((CH, 128), jnp.float32)

    pos_flat = _tc_pos(maskf_row).reshape(NP)
    deg2 = _sc_deg(dstp, ones_c)
    dis_col, dinv_col = _tc_dis(deg2)

    ysu = _tc_mlp(maskf, x, W1.astype(jnp.bfloat16), b1.reshape(1, -1),
                  W2, b2.reshape(1, -1))
    xsu_flat = _sc_sperm(ysu, pos_flat)
    xs3 = _tc_scale(xsu_flat.reshape(2, NP, 128), dis_col)
    acc1 = _sc_segsum(xs3.reshape(2 * NP, 128), srcp, dstp)
    xs2 = _tc_layer1(acc1, xs3, dis_col, dinv_col,
                     Wc1, bc1.reshape(1, -1), Wr1, br1.reshape(1, -1))
    acc2 = _sc_segsum(xs2.reshape(2 * NP, 128), srcp, dstp)
    out = _tc_layer2(acc2, xs2, dis_col, dinv_col, istj_predict,
                     Wc2, bc2.reshape(1, -1), Wr2, br2.reshape(1, -1),
                     Wf[:256], Wf[256:257], bf.reshape(1, -1))
    return out


# speedup vs baseline: 1.7295x; 1.5214x over previous
import jax
import jax.numpy as jnp
from jax import lax
from jax.experimental import pallas as pl
from jax.experimental.pallas import tpu as pltpu
from jax.experimental.pallas import tpu_sc as plsc

N = 10000
NP = 10240
E = 160000
EP = 161280
L = 16
NC = 2
NS = 16
CH = 128
PCH = 64
SEG_PER_TILE = EP // NS
DEG_PER_TILE = EP // (NC * NS)
RPT32 = NP // (NC * NS)
RPT16 = NP // NS

_MESH = plsc.VectorSubcoreMesh(core_axis_name="c", subcore_axis_name="s")


def _sc_deg_body(dst_hbm, ones_hbm, deg_out, acc_sh, ones_v, didx, zbuf):
    c = lax.axis_index("c")
    s = lax.axis_index("s")
    wid = s * NC + c

    pltpu.sync_copy(ones_hbm, ones_v)

    def zbody(i, _):
        for k in range(128 // L):
            zbuf[i, pl.ds(k * L, L)] = jnp.zeros((L,), jnp.float32)
        return 0
    lax.fori_loop(0, CH, zbody, 0)
    for t in range(RPT16 // CH):
        pltpu.sync_copy(zbuf, acc_sh.at[pl.ds(s * RPT16 + t * CH, CH)])
    plsc.subcore_barrier()

    def deg_body(j, _):
        off = wid * DEG_PER_TILE + j * 120
        pltpu.sync_copy(dst_hbm.at[pl.ds(off, 120)], didx)
        pltpu.sync_copy(ones_v, acc_sh.at[didx], add=True)
        return 0
    lax.fori_loop(0, DEG_PER_TILE // 120, deg_body, 0)
    plsc.subcore_barrier()
    pltpu.sync_copy(acc_sh.at[pl.ds(s * RPT16, RPT16)],
                    deg_out.at[c, pl.ds(s * RPT16, RPT16)])


def _sc_deg(dstp, ones_c):
    f = pl.kernel(
        _sc_deg_body,
        out_type=jax.ShapeDtypeStruct((NC, NP, 128), jnp.float32),
        mesh=_MESH,
        scratch_types=(
            pltpu.VMEM_SHARED((NP, 128), jnp.float32),
            pltpu.VMEM((120, 128), jnp.float32),
            pltpu.VMEM((120,), jnp.int32),
            pltpu.VMEM((CH, 128), jnp.float32),
        ),
    )
    return f(dstp, ones_c)


def _sc_sperm_body(ys_hbm, pos_hbm, xs_out, idx2d, rows_v, sem):
    c = lax.axis_index("c")
    s = lax.axis_index("s")
    base = s * RPT16
    nch = RPT16 // PCH
    col_off = c * NP

    for j in range(nch):
        pltpu.sync_copy(pos_hbm.at[pl.ds(base + j * PCH, PCH)], idx2d.at[j])
        for k in range(PCH // L):
            sl = pl.ds(k * L, L)
            idx2d[j, sl] = idx2d[j, sl] + col_off

    for j in range(nch):
        pltpu.sync_copy(ys_hbm.at[c, pl.ds(base + j * PCH, PCH)], rows_v)
        pltpu.async_copy(rows_v, xs_out.at[idx2d.at[j]], sem).wait()


def _sc_sperm(ys, pos_flat):
    f = pl.kernel(
        _sc_sperm_body,
        out_type=jax.ShapeDtypeStruct((2 * NP, 128), jnp.float32),
        mesh=_MESH,
        scratch_types=(
            pltpu.VMEM((RPT16 // PCH, PCH), jnp.int32),
            pltpu.VMEM((PCH, 128), jnp.float32),
            pltpu.SemaphoreType.DMA,
        ),
    )
    return f(ys, pos_flat)


_NBUF = 3
_IRING = 6
_ECH = 120
_ACCR = 10112
_NCHT = SEG_PER_TILE // _ECH


def _sc_segsum_body(xs_hbm, src2_hbm, dst_hbm, acc_out, acc_sh, *rest):
    sidx = rest[0:_IRING]
    didx = rest[_IRING:2 * _IRING]
    isem = rest[2 * _IRING:3 * _IRING]
    rows = rest[3 * _IRING:3 * _IRING + _NBUF]
    gsem = rest[3 * _IRING + _NBUF:3 * _IRING + 2 * _NBUF]
    ssem = rest[3 * _IRING + 2 * _NBUF:3 * _IRING + 3 * _NBUF]
    c = lax.axis_index("c")
    s = lax.axis_index("s")
    ebase = s * SEG_PER_TILE

    def ifire(j):
        q = j % _IRING
        off = c * EP + ebase + j * _ECH
        pltpu.async_copy(src2_hbm.at[pl.ds(off, _ECH)], sidx[q], isem[q])
        pltpu.async_copy(dst_hbm.at[pl.ds(ebase + j * _ECH, _ECH)],
                         didx[q], isem[q])

    def iwait(j):
        q = j % _IRING
        off = c * EP + ebase + j * _ECH
        pltpu.make_async_copy(src2_hbm.at[pl.ds(off, _ECH)], sidx[q],
                              isem[q]).wait()
        pltpu.make_async_copy(dst_hbm.at[pl.ds(ebase + j * _ECH, _ECH)],
                              didx[q], isem[q]).wait()

    def zbody(i, _):
        for b in range(_NBUF):
            for k in range(128 // L):
                rows[b][i, pl.ds(k * L, L)] = jnp.zeros((L,), jnp.float32)
        return 0
    lax.fori_loop(0, _ECH, zbody, 0)
    abase = s * (_ACCR // NS)
    for t in range(5):
        pltpu.sync_copy(rows[0], acc_sh.at[pl.ds(abase + t * _ECH, _ECH)])
    pltpu.sync_copy(rows[0].at[pl.ds(0, 32)],
                    acc_sh.at[pl.ds(abase + 5 * _ECH, 32)])
    plsc.subcore_barrier()

    for j in range(_NBUF):
        ifire(j)
    for j in range(_NBUF):
        iwait(j)
    for b in range(_NBUF):
        pltpu.async_copy(rows[b], acc_sh.at[didx[b]], ssem[b], add=True)

    for j in range(_NCHT):
        b = j % _NBUF
        q = j % _IRING
        pltpu.make_async_copy(rows[b], acc_sh.at[didx[0]], ssem[b]).wait()
        if j >= _NBUF:
            iwait(j)
        pltpu.async_copy(xs_hbm.at[sidx[q]], rows[b], gsem[b])
        if j + _NBUF < _NCHT:
            ifire(j + _NBUF)
        if j >= 2:
            jj = j - 2
            bb = jj % _NBUF
            qq = jj % _IRING
            pltpu.make_async_copy(xs_hbm.at[sidx[0]], rows[bb],
                                  gsem[bb]).wait()
            pltpu.async_copy(rows[bb], acc_sh.at[didx[qq]], ssem[bb],
                             add=True)
    for jj in range(_NCHT - 2, _NCHT):
        bb = jj % _NBUF
        qq = jj % _IRING
        pltpu.make_async_copy(xs_hbm.at[sidx[0]], rows[bb], gsem[bb]).wait()
        pltpu.async_copy(rows[bb], acc_sh.at[didx[qq]], ssem[bb], add=True)
    for b in range(_NBUF):
        pltpu.make_async_copy(rows[b], acc_sh.at[didx[0]], ssem[b]).wait()
    plsc.subcore_barrier()

    for t in range(5):
        pltpu.sync_copy(acc_sh.at[pl.ds(abase + t * _ECH, _ECH)],
                        acc_out.at[c, pl.ds(abase + t * _ECH, _ECH)])
    pltpu.sync_copy(acc_sh.at[pl.ds(abase + 5 * _ECH, 32)],
                    acc_out.at[c, pl.ds(abase + 5 * _ECH, 32)])


def _sc_segsum(xs_flat, src2, dstp):
    f = pl.kernel(
        _sc_segsum_body,
        out_type=jax.ShapeDtypeStruct((2, NP, 128), jnp.float32),
        mesh=_MESH,
        scratch_types=(
            (pltpu.VMEM_SHARED((_ACCR, 128), jnp.float32),)
            + tuple(pltpu.VMEM((_ECH,), jnp.int32) for _ in range(_IRING))
            + tuple(pltpu.VMEM((_ECH,), jnp.int32) for _ in range(_IRING))
            + tuple(pltpu.SemaphoreType.DMA for _ in range(_IRING))
            + tuple(pltpu.VMEM((_ECH, 128), jnp.float32)
                    for _ in range(_NBUF))
            + tuple(pltpu.SemaphoreType.DMA for _ in range(_NBUF))
            + tuple(pltpu.SemaphoreType.DMA for _ in range(_NBUF))
        ),
    )
    return f(xs_flat, src2, dstp)


_PC = 1024


def _tc_pos_body(mask_ref, pos_ref, carry):
    p = pl.program_id(0)
    j = pl.program_id(1)

    @pl.when(jnp.logical_and(p == 0, j == 0))
    def _():
        carry[0] = 0.0

    m = mask_ref[...]
    msum = jnp.sum(m)

    @pl.when(p == 0)
    def _():
        carry[0] = carry[0] + msum
        pos_ref[...] = jnp.zeros((1, _PC), jnp.int32)

    @pl.when(p == 1)
    def _():
        @pl.when(j == 0)
        def _():
            carry[1] = 0.0
        r = lax.broadcasted_iota(jnp.int32, (_PC, _PC), 0)
        cc = lax.broadcasted_iota(jnp.int32, (_PC, _PC), 1)
        ut = (r <= cc).astype(jnp.float32)
        incl = jnp.dot(m, ut, preferred_element_type=jnp.float32)
        excl = incl - m
        cum_t = carry[1] + excl
        ii = (lax.broadcasted_iota(jnp.int32, (1, _PC), 1).astype(jnp.float32)
              + jnp.float32(_PC) * j.astype(jnp.float32))
        k_tot = carry[0]
        posf = jnp.where(m > 0.5, cum_t, k_tot + ii - cum_t)
        pos_ref[...] = posf.astype(jnp.int32)
        carry[1] = carry[1] + msum


def _tc_pos(maskf_row):
    return pl.pallas_call(
        _tc_pos_body,
        grid=(2, NP // _PC),
        in_specs=[pl.BlockSpec((1, _PC), lambda p, j: (0, j))],
        out_specs=pl.BlockSpec((1, _PC), lambda p, j: (0, j)),
        out_shape=jax.ShapeDtypeStruct((1, NP), jnp.int32),
        scratch_shapes=[pltpu.SMEM((2,), jnp.float32)],
        compiler_params=pltpu.CompilerParams(
            dimension_semantics=("arbitrary", "arbitrary")),
    )(maskf_row)


_DR = 1280


def _tc_dis_body(deg_ref, dis_ref, dinv_ref):
    d = deg_ref[0, :, 0:1] + deg_ref[1, :, 0:1] + 1.0
    dis_ref[...] = lax.rsqrt(d)
    dinv_ref[...] = jnp.sqrt(d)


def _tc_dis(deg2):
    return pl.pallas_call(
        _tc_dis_body,
        grid=(NP // _DR,),
        in_specs=[pl.BlockSpec((2, _DR, 128), lambda i: (0, i, 0))],
        out_specs=(pl.BlockSpec((_DR, 1), lambda i: (i, 0)),
                   pl.BlockSpec((_DR, 1), lambda i: (i, 0))),
        out_shape=(jax.ShapeDtypeStruct((NP, 1), jnp.float32),
                   jax.ShapeDtypeStruct((NP, 1), jnp.float32)),
        compiler_params=pltpu.CompilerParams(
            dimension_semantics=("parallel",)),
    )(deg2)


def _tc_scale_body(xsu_ref, dis_ref, xs_ref):
    d = dis_ref[...]
    xs_ref[0] = xsu_ref[0] * d
    xs_ref[1] = xsu_ref[1] * d


def _tc_scale(xsu3, dis_col):
    return pl.pallas_call(
        _tc_scale_body,
        grid=(N // _LR,),
        in_specs=[pl.BlockSpec((2, _LR, 128), lambda i: (0, i, 0)),
                  pl.BlockSpec((_LR, 1), lambda i: (i, 0))],
        out_specs=pl.BlockSpec((2, _LR, 128), lambda i: (0, i, 0)),
        out_shape=jax.ShapeDtypeStruct((2, NP, 128), jnp.float32),
        compiler_params=pltpu.CompilerParams(
            dimension_semantics=("parallel",)),
    )(xsu3, dis_col)


_MR = 1000
_MK = 512
_NKT = 4096 // _MK


def _tc_mlp_body(maskf_ref, x_ref, w1_ref, b1_ref, w2_ref, b2_ref,
                 ys_ref, acc_s, gene_s):
    k = pl.program_id(1)

    @pl.when(k == 0)
    def _():
        gene_s[...] = x_ref[:, :256]

    prod = jnp.dot(x_ref[...].astype(jnp.bfloat16), w1_ref[...],
                   preferred_element_type=jnp.float32)

    @pl.when(k == 0)
    def _():
        acc_s[...] = prod

    @pl.when(k > 0)
    def _():
        acc_s[...] = acc_s[...] + prod

    @pl.when(k == _NKT - 1)
    def _():
        g = jnp.maximum(acc_s[...] + b1_ref[...], 0.0)
        g = jnp.maximum(jnp.dot(g, w2_ref[...],
                                preferred_element_type=jnp.float32)
                        + b2_ref[...], 0.0)
        ys = jnp.where(maskf_ref[...] > 0.5, gene_s[...], g)
        ys_ref[0] = ys[:, :128]
        ys_ref[1] = ys[:, 128:]


def _tc_mlp(maskf, x, w1, b1r, w2, b2r):
    return pl.pallas_call(
        _tc_mlp_body,
        grid=(N // _MR, _NKT),
        in_specs=[
            pl.BlockSpec((_MR, 1), lambda i, k: (i, 0)),
            pl.BlockSpec((_MR, _MK), lambda i, k: (i, k)),
            pl.BlockSpec((_MK, 1024), lambda i, k: (k, 0)),
            pl.BlockSpec((1, 1024), lambda i, k: (0, 0)),
            pl.BlockSpec((1024, 256), lambda i, k: (0, 0)),
            pl.BlockSpec((1, 256), lambda i, k: (0, 0)),
        ],
        out_specs=pl.BlockSpec((2, _MR, 128), lambda i, k: (0, i, 0)),
        out_shape=jax.ShapeDtypeStruct((2, NP, 128), jnp.float32),
        scratch_shapes=[pltpu.VMEM((_MR, 1024), jnp.float32),
                        pltpu.VMEM((_MR, 256), jnp.float32)],
        compiler_params=pltpu.CompilerParams(
            dimension_semantics=("parallel", "arbitrary")),
    )(maskf, x, w1, b1r, w2, b2r)


_LR = 1000


def _tc_l1_body(acc_ref, xs_ref, dis_ref, dinv_ref,
                wc_ref, bc_ref, wr_ref, br_ref, out_ref):
    xlo = xs_ref[0]
    xhi = xs_ref[1]
    dis = dis_ref[...]
    agg = jnp.concatenate([acc_ref[0] + xlo, acc_ref[1] + xhi], axis=1) * dis
    xt = jnp.concatenate([xlo, xhi], axis=1) * dinv_ref[...]
    h = jnp.maximum(jnp.dot(agg, wc_ref[...],
                            preferred_element_type=jnp.float32)
                    + bc_ref[...], 0.0)
    h = h + jnp.dot(xt, wr_ref[...], preferred_element_type=jnp.float32) \
        + br_ref[...]
    xs2 = h * dis
    out_ref[0] = xs2[:, :128]
    out_ref[1] = xs2[:, 128:]


def _tc_layer1(acc3, xs3, dis_col, dinv_col, wc, bcr, wr, brr):
    return pl.pallas_call(
        _tc_l1_body,
        grid=(N // _LR,),
        in_specs=[
            pl.BlockSpec((2, _LR, 128), lambda i: (0, i, 0)),
            pl.BlockSpec((2, _LR, 128), lambda i: (0, i, 0)),
            pl.BlockSpec((_LR, 1), lambda i: (i, 0)),
            pl.BlockSpec((_LR, 1), lambda i: (i, 0)),
            pl.BlockSpec((256, 256), lambda i: (0, 0)),
            pl.BlockSpec((1, 256), lambda i: (0, 0)),
            pl.BlockSpec((256, 256), lambda i: (0, 0)),
            pl.BlockSpec((1, 256), lambda i: (0, 0)),
        ],
        out_specs=pl.BlockSpec((2, _LR, 128), lambda i: (0, i, 0)),
        out_shape=jax.ShapeDtypeStruct((2, NP, 128), jnp.float32),
        compiler_params=pltpu.CompilerParams(
            dimension_semantics=("parallel",)),
    )(acc3, xs3, dis_col, dinv_col, wc, bcr, wr, brr)


def _tc_l2_body(acc_ref, xs_ref, dis_ref, dinv_ref, istj_ref,
                wc_ref, bc_ref, wr_ref, br_ref, wf_ref, wfl_ref, bf_ref,
                out_ref):
    xlo = xs_ref[0]
    xhi = xs_ref[1]
    agg = jnp.concatenate([acc_ref[0] + xlo, acc_ref[1] + xhi],
                          axis=1) * dis_ref[...]
    h1 = jnp.concatenate([xlo, xhi], axis=1) * dinv_ref[...]
    h2 = jnp.maximum(jnp.dot(agg, wc_ref[...],
                             preferred_element_type=jnp.float32)
                     + bc_ref[...], 0.0)
    h2 = h2 + jnp.dot(h1, wr_ref[...], preferred_element_type=jnp.float32) \
        + br_ref[...]
    out_ref[...] = (jnp.dot(h2, wf_ref[...],
                            preferred_element_type=jnp.float32)
                    + istj_ref[...] * wfl_ref[...] + bf_ref[...])


def _tc_layer2(acc3, xs3, dis_col, dinv_col, istj,
               wc, bcr, wr, brr, wf256, wflast, bfr):
    return pl.pallas_call(
        _tc_l2_body,
        grid=(N // _LR,),
        in_specs=[
            pl.BlockSpec((2, _LR, 128), lambda i: (0, i, 0)),
            pl.BlockSpec((2, _LR, 128), lambda i: (0, i, 0)),
            pl.BlockSpec((_LR, 1), lambda i: (i, 0)),
            pl.BlockSpec((_LR, 1), lambda i: (i, 0)),
            pl.BlockSpec((_LR, 1), lambda i: (i, 0)),
            pl.BlockSpec((256, 256), lambda i: (0, 0)),
            pl.BlockSpec((1, 256), lambda i: (0, 0)),
            pl.BlockSpec((256, 256), lambda i: (0, 0)),
            pl.BlockSpec((1, 256), lambda i: (0, 0)),
            pl.BlockSpec((256, 128), lambda i: (0, 0)),
            pl.BlockSpec((1, 128), lambda i: (0, 0)),
            pl.BlockSpec((1, 128), lambda i: (0, 0)),
        ],
        out_specs=pl.BlockSpec((_LR, 128), lambda i: (i, 0)),
        out_shape=jax.ShapeDtypeStruct((N, 128), jnp.float32),
        compiler_params=pltpu.CompilerParams(
            dimension_semantics=("parallel",)),
    )(acc3, xs3, dis_col, dinv_col, istj, wc, bcr, wr, brr,
      wf256, wflast, bfr)


def kernel(x, edge_index, mask, istj_predict, W1, b1, W2, b2,
           Wc1, bc1, Wc2, bc2, Wr1, br1, Wr2, br2, Wf, bf):
    maskf = mask.astype(jnp.float32).reshape(N, 1)
    maskf_row = jnp.pad(mask.astype(jnp.float32), (0, NP - N)).reshape(1, NP)
    src = edge_index[0]
    dst = edge_index[1]
    srcp = jnp.concatenate([src, jnp.full((EP - E,), N, jnp.int32)])
    dstp = jnp.concatenate([dst, jnp.full((EP - E,), 10104, jnp.int32)])
    src2 = jnp.concatenate([srcp, srcp + NP])
    ones_c = jnp.ones((120, 128), jnp.float32)

    pos_flat = _tc_pos(maskf_row).reshape(NP)
    deg2 = _sc_deg(dstp, ones_c)
    dis_col, dinv_col = _tc_dis(deg2)

    ysu = _tc_mlp(maskf, x, W1.astype(jnp.bfloat16), b1.reshape(1, -1),
                  W2, b2.reshape(1, -1))
    xsu_flat = _sc_sperm(ysu, pos_flat)
    xs3 = _tc_scale(xsu_flat.reshape(2, NP, 128), dis_col)
    acc1 = _sc_segsum(xs3.reshape(2 * NP, 128), src2, dstp)
    xs2 = _tc_layer1(acc1, xs3, dis_col, dinv_col,
                     Wc1, bc1.reshape(1, -1), Wr1, br1.reshape(1, -1))
    acc2 = _sc_segsum(xs2.reshape(2 * NP, 128), src2, dstp)
    out = _tc_layer2(acc2, xs2, dis_col, dinv_col, istj_predict,
                     Wc2, bc2.reshape(1, -1), Wr2, br2.reshape(1, -1),
                     Wf[:256], Wf[256:257], bf.reshape(1, -1))
    return out
